# baseline (speedup 1.0000x reference)
import functools
import math

import jax
import jax.numpy as jnp
from jax import lax
from jax.experimental import pallas as pl
from jax.experimental.pallas import tpu as pltpu
from jax.experimental.pallas import tpu_sc as plsc

D = 64
SCALE = math.sqrt(D)
CHUNK = 128
NBUF = 4
GROUP_ROWS = NBUF * CHUNK
LANES = 16


def _sc_kernel(num_chunks_per_worker, nc, ns):
  ngroups = num_chunks_per_worker // NBUF
  mesh = plsc.VectorSubcoreMesh(core_axis_name="c", subcore_axis_name="s")

  scratch = [pltpu.VMEM((num_chunks_per_worker, CHUNK), jnp.int32)]
  scratch += [pltpu.VMEM((CHUNK, D), jnp.float32) for _ in range(NBUF)]
  scratch += [pltpu.VMEM((GROUP_ROWS, D), jnp.float32) for _ in range(2)]
  scratch += [pltpu.SemaphoreType.DMA for _ in range(NBUF + 2)]

  total_rows = num_chunks_per_worker * CHUNK * nc * ns

  @functools.partial(
      pl.kernel,
      out_type=jax.ShapeDtypeStruct((total_rows, D), jnp.float32),
      mesh=mesh,
      scratch_types=scratch,
      compiler_params=pltpu.CompilerParams(use_tc_tiling_on_sc=False),
  )
  def k(idx_hbm, table_hbm, out_hbm, idx_v, *rest):
    g_bufs = rest[:NBUF]
    s_bufs = rest[NBUF:NBUF + 2]
    gsem = rest[NBUF + 2:2 * NBUF + 2]
    ssem = rest[2 * NBUF + 2:]
    wid = lax.axis_index("s") * nc + lax.axis_index("c")
    base_chunk = wid * num_chunks_per_worker

    pltpu.sync_copy(idx_hbm.at[pl.ds(base_chunk, num_chunks_per_worker)],
                    idx_v)

    def gather_start(j, b):
      pltpu.async_copy(table_hbm.at[idx_v.at[j]], g_bufs[b], gsem[b])

    def gather_wait(j, b):
      pltpu.make_async_copy(table_hbm.at[idx_v.at[j]], g_bufs[b],
                            gsem[b]).wait()

    def scatter_start(g, p):
      dst = out_hbm.at[pl.ds((base_chunk + g * NBUF) * CHUNK, GROUP_ROWS)]
      pltpu.async_copy(s_bufs[p], dst, ssem[p])

    def scatter_wait(g, p):
      dst = out_hbm.at[pl.ds((base_chunk + g * NBUF) * CHUNK, GROUP_ROWS)]
      pltpu.make_async_copy(s_bufs[p], dst, ssem[p]).wait()

    def scale_into(b, p):
      src = g_bufs[b]
      dst = s_bufs[p]

      @pl.loop(0, CHUNK, unroll=4)
      def _(r):
        for c in range(D // LANES):
          sl = pl.ds(c * LANES, LANES)
          dst[r + b * CHUNK, sl] = src[r, sl] * SCALE

    def group_body(g, p, start_next, wait_prev):
      for b in range(NBUF):
        j = g * NBUF + b
        gather_wait(j, b)
        scale_into(b, p)
        if start_next:
          gather_start(j + NBUF, b)
      if wait_prev:
        scatter_wait(g - 2, p)
      scatter_start(g, p)

    for b in range(NBUF):
      gather_start(b, b)

    group_body(0, 0, True, False)
    group_body(1, 1, True, False)

    @pl.loop(1, ngroups // 2 - 1)
    def _(h):
      group_body(2 * h, 0, True, True)
      group_body(2 * h + 1, 1, True, True)

    group_body(ngroups - 2, 0, True, True)
    group_body(ngroups - 1, 1, False, True)

    scatter_wait(ngroups - 2, (ngroups - 2) % 2)
    scatter_wait(ngroups - 1, (ngroups - 1) % 2)

  return k


def kernel(x, table):
  xs, ts = x.shape, table.shape
  b_total = xs[0] * xs[1]
  info = plsc.get_sparse_core_info()
  nw = info.num_cores * info.num_subcores
  num_chunks_per_worker = b_total // (CHUNK * nw)
  idx = jnp.reshape(x.astype(jnp.int32), (b_total // CHUNK, CHUNK))
  k = _sc_kernel(num_chunks_per_worker, info.num_cores, info.num_subcores)
  out = k(idx, table)
  return jnp.reshape(out, (xs[0], xs[1], ts[1]))


# speedup vs baseline: 1.6439x; 1.6439x over previous
import functools
import math

import jax
import jax.numpy as jnp
from jax import lax
from jax.experimental import pallas as pl
from jax.experimental.pallas import tpu as pltpu
from jax.experimental.pallas import tpu_sc as plsc

D = 64
WIDE = 128
SCALE = math.sqrt(D)
CHUNK = 128
NBUF = 4
VBLK = 4096


def _widen_kernel(tt_ref, out_ref):
  blk = tt_ref[...].astype(jnp.float32)
  out_ref[:, 0:D] = blk.T * SCALE
  out_ref[:, D:WIDE] = jnp.zeros((VBLK, WIDE - D), jnp.float32)


def _widen(table_t):
  vocab = table_t.shape[1]
  grid = (pl.cdiv(vocab, VBLK),)
  return pl.pallas_call(
      _widen_kernel,
      out_shape=jax.ShapeDtypeStruct((vocab, WIDE), jnp.float32),
      grid=grid,
      in_specs=[pl.BlockSpec((D, VBLK), lambda i: (0, i))],
      out_specs=pl.BlockSpec((VBLK, WIDE), lambda i: (i, 0)),
  )(table_t)


def _sc_kernel(num_chunks_per_worker, nc, ns, vocab):
  ngroups = num_chunks_per_worker // NBUF
  mesh = plsc.VectorSubcoreMesh(core_axis_name="c", subcore_axis_name="s")

  scratch = [pltpu.VMEM((num_chunks_per_worker, CHUNK), jnp.int32)]
  scratch += [pltpu.VMEM((CHUNK, WIDE), jnp.float32) for _ in range(NBUF)]
  scratch += [pltpu.SemaphoreType.DMA for _ in range(2 * NBUF)]

  total_rows = num_chunks_per_worker * CHUNK * nc * ns

  @functools.partial(
      pl.kernel,
      out_type=jax.ShapeDtypeStruct((total_rows, WIDE), jnp.float32),
      mesh=mesh,
      scratch_types=scratch,
  )
  def k(idx_hbm, table_hbm, out_hbm, idx_v, *rest):
    rows = rest[:NBUF]
    gsem = rest[NBUF:2 * NBUF]
    ssem = rest[2 * NBUF:]
    wid = lax.axis_index("s") * nc + lax.axis_index("c")
    base_chunk = wid * num_chunks_per_worker

    pltpu.sync_copy(idx_hbm.at[pl.ds(base_chunk, num_chunks_per_worker)],
                    idx_v)

    def gather_start(j, b):
      pltpu.async_copy(table_hbm.at[idx_v.at[j]], rows[b], gsem[b])

    def gather_wait(j, b):
      pltpu.make_async_copy(table_hbm.at[idx_v.at[j]], rows[b],
                            gsem[b]).wait()

    def scatter_start(j, b):
      dst = out_hbm.at[pl.ds((base_chunk + j) * CHUNK, CHUNK)]
      pltpu.async_copy(rows[b], dst, ssem[b])

    def scatter_wait(j, b):
      dst = out_hbm.at[pl.ds((base_chunk + j) * CHUNK, CHUNK)]
      pltpu.make_async_copy(rows[b], dst, ssem[b]).wait()

    for b in range(NBUF):
      gather_start(b, b)

    @pl.loop(0, ngroups - 1)
    def _(g):
      for b in range(NBUF):
        j = g * NBUF + b
        gather_wait(j, b)
        scatter_start(j, b)
        scatter_wait(j, b)
        gather_start(j + NBUF, b)

    for b in range(NBUF):
      j = (ngroups - 1) * NBUF + b
      gather_wait(j, b)
      scatter_start(j, b)
      scatter_wait(j, b)

  return k


def kernel(x, table):
  xs, ts = x.shape, table.shape
  vocab = ts[0]
  b_total = xs[0] * xs[1]
  info = plsc.get_sparse_core_info()
  nw = info.num_cores * info.num_subcores
  num_chunks_per_worker = b_total // (CHUNK * nw)
  idx = jnp.reshape(x.astype(jnp.int32), (b_total // CHUNK, CHUNK))
  twide = _widen(table.T)
  k = _sc_kernel(num_chunks_per_worker, info.num_cores, info.num_subcores,
                 vocab)
  out = k(idx, twide)
  return jnp.reshape(out[:, :D], (xs[0], xs[1], ts[1]))
